# Initial kernel scaffold; baseline (speedup 1.0000x reference)
import jax
import jax.numpy as jnp
from jax.experimental import pallas as pl


def kernel(word_ids, ml, f, lf, ll, edge_index, emb_table, W1, b1, W2, b2):
    raise NotImplementedError("write your pallas kernel here")


# speedup vs baseline: 1.8502x; 1.8502x over previous
import jax
import jax.numpy as jnp
from jax import lax
from jax.experimental import pallas as pl
from jax.experimental.pallas import tpu as pltpu
from jax.experimental.pallas import tpu_sc as plsc

N = 10000
E = 160000
L = 20
V = 50000
D = 256
Q = 64

NP = 10240
NB = NP // 16
WPT = NB * L
EPT = 10112
EP = EPT * 16
C = 128
TRASH = NP - 1

_mesh = plsc.VectorSubcoreMesh(core_axis_name="c", subcore_axis_name="s")


def _embdeg_body(t0, t1, t2, t3, wids, nids, esrc, edst, z64, z16, ones16,
                 we0, we1, we2, we3, dsrc, ddst,
                 widx, nidx, eidx, rows, obuf, vbuf, vbuf16, acc, dacc, sem):
    c = lax.axis_index("c")
    s = lax.axis_index("s")
    base_n = s * NB
    base_w = s * WPT
    pltpu.sync_copy(z64, vbuf)
    pltpu.sync_copy(z16, vbuf16)
    pltpu.sync_copy(ones16, obuf)
    pltpu.sync_copy(vbuf16, dacc.at[pl.ds(base_n, NB)])

    for r, (ta, tb, oa, ob) in enumerate(((t0, t2, we0, we2),
                                          (t1, t3, we1, we3))):
        pltpu.sync_copy(vbuf, acc.at[pl.ds(base_n, NB)])
        plsc.subcore_barrier()

        def emb_step(k, carry):
            off = base_w + k * C
            pltpu.sync_copy(wids.at[pl.ds(off, C)], widx)
            pltpu.sync_copy(nids.at[pl.ds(off, C)], nidx)

            @pl.when(c == 0)
            def _():
                pltpu.async_copy(ta.at[widx], rows, sem).wait()

            @pl.when(c == 1)
            def _():
                pltpu.async_copy(tb.at[widx], rows, sem).wait()

            pltpu.sync_copy(rows, acc.at[nidx], add=True)
            return carry

        lax.fori_loop(0, WPT // C, emb_step, 0)
        plsc.subcore_barrier()
        pltpu.sync_copy(acc.at[pl.ds(base_n, NB)], vbuf)

        @pl.when(c == 0)
        def _():
            pltpu.sync_copy(vbuf, oa.at[pl.ds(base_n, NB)])

        @pl.when(c == 1)
        def _():
            pltpu.sync_copy(vbuf, ob.at[pl.ds(base_n, NB)])

        if r == 0:
            pltpu.sync_copy(z64, vbuf)

    base_e = s * EPT

    def deg_step(k, carry):
        off = base_e + k * C

        @pl.when(c == 0)
        def _():
            pltpu.sync_copy(esrc.at[pl.ds(off, C)], eidx)

        @pl.when(c == 1)
        def _():
            pltpu.sync_copy(edst.at[pl.ds(off, C)], eidx)

        pltpu.sync_copy(obuf, dacc.at[eidx], add=True)
        return carry

    lax.fori_loop(0, EPT // C, deg_step, 0)

    plsc.subcore_barrier()
    pltpu.sync_copy(dacc.at[pl.ds(base_n, NB)], vbuf16)

    @pl.when(c == 0)
    def _():
        pltpu.sync_copy(vbuf16, dsrc.at[pl.ds(base_n, NB)])

    @pl.when(c == 1)
    def _():
        pltpu.sync_copy(vbuf16, ddst.at[pl.ds(base_n, NB)])


_embdeg = pl.kernel(
    _embdeg_body,
    out_type=(
        jax.ShapeDtypeStruct((NP, Q), jnp.float32),
        jax.ShapeDtypeStruct((NP, Q), jnp.float32),
        jax.ShapeDtypeStruct((NP, Q), jnp.float32),
        jax.ShapeDtypeStruct((NP, Q), jnp.float32),
        jax.ShapeDtypeStruct((NP, 16), jnp.float32),
        jax.ShapeDtypeStruct((NP, 16), jnp.float32),
    ),
    mesh=_mesh,
    compiler_params=pltpu.CompilerParams(use_tc_tiling_on_sc=False),
    scratch_types=[
        pltpu.VMEM((C,), jnp.int32),
        pltpu.VMEM((C,), jnp.int32),
        pltpu.VMEM((C,), jnp.int32),
        pltpu.VMEM((C, Q), jnp.float32),
        pltpu.VMEM((C, 16), jnp.float32),
        pltpu.VMEM((NB, Q), jnp.float32),
        pltpu.VMEM((NB, 16), jnp.float32),
        pltpu.VMEM_SHARED((NP, Q), jnp.float32),
        pltpu.VMEM_SHARED((NP, 16), jnp.float32),
        pltpu.SemaphoreType.DMA,
    ],
)


def _msg_body(x0, x1, x2, x3, esrc, edst, z64,
              m0, m1, m2, m3,
              sidx, didx, rows, vbuf, acc, sem):
    c = lax.axis_index("c")
    s = lax.axis_index("s")
    base_n = s * NB
    base_e = s * EPT
    pltpu.sync_copy(z64, vbuf)

    for r, (xa, xb, oa, ob) in enumerate(((x0, x2, m0, m2),
                                          (x1, x3, m1, m3))):
        pltpu.sync_copy(vbuf, acc.at[pl.ds(base_n, NB)])
        plsc.subcore_barrier()

        def step(k, carry):
            off = base_e + k * C
            pltpu.sync_copy(esrc.at[pl.ds(off, C)], sidx)
            pltpu.sync_copy(edst.at[pl.ds(off, C)], didx)

            @pl.when(c == 0)
            def _():
                pltpu.async_copy(xa.at[sidx], rows, sem).wait()

            @pl.when(c == 1)
            def _():
                pltpu.async_copy(xb.at[sidx], rows, sem).wait()

            pltpu.sync_copy(rows, acc.at[didx], add=True)
            return carry

        lax.fori_loop(0, EPT // C, step, 0)
        plsc.subcore_barrier()
        pltpu.sync_copy(acc.at[pl.ds(base_n, NB)], vbuf)

        @pl.when(c == 0)
        def _():
            pltpu.sync_copy(vbuf, oa.at[pl.ds(base_n, NB)])

        @pl.when(c == 1)
        def _():
            pltpu.sync_copy(vbuf, ob.at[pl.ds(base_n, NB)])

        if r == 0:
            pltpu.sync_copy(z64, vbuf)


_msg = pl.kernel(
    _msg_body,
    out_type=(
        jax.ShapeDtypeStruct((NP, Q), jnp.float32),
        jax.ShapeDtypeStruct((NP, Q), jnp.float32),
        jax.ShapeDtypeStruct((NP, Q), jnp.float32),
        jax.ShapeDtypeStruct((NP, Q), jnp.float32),
    ),
    mesh=_mesh,
    compiler_params=pltpu.CompilerParams(use_tc_tiling_on_sc=False),
    scratch_types=[
        pltpu.VMEM((C,), jnp.int32),
        pltpu.VMEM((C,), jnp.int32),
        pltpu.VMEM((C, Q), jnp.float32),
        pltpu.VMEM((NB, Q), jnp.float32),
        pltpu.VMEM_SHARED((NP, Q), jnp.float32),
        pltpu.SemaphoreType.DMA,
    ],
)

B = 640
GRID = NP // B


def _tc1_body(we0_ref, we1_ref, we2_ref, we3_ref, scal_ref, dsrc_ref, w_ref,
              o0_ref, o1_ref, o2_ref, o3_ref):
    h = jnp.concatenate([we0_ref[...], we1_ref[...],
                         we2_ref[...], we3_ref[...]], axis=1)
    ml = scal_ref[:, 0:1]
    f = scal_ref[:, 1:2]
    lf = scal_ref[:, 2:3]
    ll = scal_ref[:, 3:4]
    col = lax.broadcasted_iota(jnp.int32, (B, D), 1)
    h = h / ml
    h = jnp.where(col == D - 3, f, h)
    h = jnp.where(col == D - 2, lf, h)
    h = jnp.where(col == D - 1, ll, h)
    sout = lax.rsqrt(jnp.maximum(dsrc_ref[:, 0:1], 1.0))
    x = jnp.dot(h * sout, w_ref[...], preferred_element_type=jnp.float32)
    o0_ref[...] = x[:, 0 * Q:1 * Q]
    o1_ref[...] = x[:, 1 * Q:2 * Q]
    o2_ref[...] = x[:, 2 * Q:3 * Q]
    o3_ref[...] = x[:, 3 * Q:4 * Q]


_quarter_spec = pl.BlockSpec((B, Q), lambda i: (i, 0))

_tc1 = pl.pallas_call(
    _tc1_body,
    grid=(GRID,),
    in_specs=[
        _quarter_spec, _quarter_spec, _quarter_spec, _quarter_spec,
        pl.BlockSpec((B, 4), lambda i: (i, 0)),
        pl.BlockSpec((B, 16), lambda i: (i, 0)),
        pl.BlockSpec((D, D), lambda i: (0, 0)),
    ],
    out_specs=[_quarter_spec, _quarter_spec, _quarter_spec, _quarter_spec],
    out_shape=[jax.ShapeDtypeStruct((NP, Q), jnp.float32)] * 4,
)


def _tc2_body(m0_ref, m1_ref, m2_ref, m3_ref, dsrc_ref, ddst_ref, b_ref,
              w_ref, o0_ref, o1_ref, o2_ref, o3_ref):
    agg = jnp.concatenate([m0_ref[...], m1_ref[...],
                           m2_ref[...], m3_ref[...]], axis=1)
    sin = lax.rsqrt(jnp.maximum(ddst_ref[:, 0:1], 1.0))
    sout = lax.rsqrt(jnp.maximum(dsrc_ref[:, 0:1], 1.0))
    h = jnp.maximum(agg * sin + b_ref[...], 0.0)
    x = jnp.dot(h * sout, w_ref[...], preferred_element_type=jnp.float32)
    o0_ref[...] = x[:, 0 * Q:1 * Q]
    o1_ref[...] = x[:, 1 * Q:2 * Q]
    o2_ref[...] = x[:, 2 * Q:3 * Q]
    o3_ref[...] = x[:, 3 * Q:4 * Q]


_tc2 = pl.pallas_call(
    _tc2_body,
    grid=(GRID,),
    in_specs=[
        _quarter_spec, _quarter_spec, _quarter_spec, _quarter_spec,
        pl.BlockSpec((B, 16), lambda i: (i, 0)),
        pl.BlockSpec((B, 16), lambda i: (i, 0)),
        pl.BlockSpec((1, D), lambda i: (0, 0)),
        pl.BlockSpec((D, D), lambda i: (0, 0)),
    ],
    out_specs=[_quarter_spec, _quarter_spec, _quarter_spec, _quarter_spec],
    out_shape=[jax.ShapeDtypeStruct((NP, Q), jnp.float32)] * 4,
)


def _tc3_body(m0_ref, m1_ref, m2_ref, m3_ref, ddst_ref, b_ref,
              oh_ref, og_ref):
    agg = jnp.concatenate([m0_ref[...], m1_ref[...],
                           m2_ref[...], m3_ref[...]], axis=1)
    sin = lax.rsqrt(jnp.maximum(ddst_ref[:, 0:1], 1.0))
    h = jnp.maximum(agg * sin + b_ref[...], 0.0)
    oh_ref[...] = h
    i = pl.program_id(0)
    rows = lax.broadcasted_iota(jnp.int32, (B, 1), 0) + i * B
    part = jnp.sum(jnp.where(rows < N, h, 0.0), axis=0, keepdims=True)

    @pl.when(i == 0)
    def _():
        og_ref[...] = jnp.zeros_like(og_ref)

    og_ref[...] += part

    @pl.when(i == GRID - 1)
    def _():
        og_ref[...] = og_ref[...] * (1.0 / N)


_tc3 = pl.pallas_call(
    _tc3_body,
    grid=(GRID,),
    in_specs=[
        _quarter_spec, _quarter_spec, _quarter_spec, _quarter_spec,
        pl.BlockSpec((B, 16), lambda i: (i, 0)),
        pl.BlockSpec((1, D), lambda i: (0, 0)),
    ],
    out_specs=[
        pl.BlockSpec((B, D), lambda i: (i, 0)),
        pl.BlockSpec((1, D), lambda i: (0, 0)),
    ],
    out_shape=[
        jax.ShapeDtypeStruct((NP, D), jnp.float32),
        jax.ShapeDtypeStruct((1, D), jnp.float32),
    ],
)


@jax.jit
def kernel(word_ids, ml, f, lf, ll, edge_index, emb_table, W1, b1, W2, b2):
    t = jnp.pad(emb_table, ((0, 0), (0, D - emb_table.shape[1])))
    tq = [t[:, i * Q:(i + 1) * Q] for i in range(4)]
    wpad = jnp.concatenate(
        [word_ids.astype(jnp.int32).reshape(-1),
         jnp.ones(((NP - N) * L,), jnp.int32)])
    nids = jnp.arange(NP * L, dtype=jnp.int32) // L
    ei = edge_index.astype(jnp.int32)
    pad_e = jnp.full((EP - E,), TRASH, jnp.int32)
    esrc = jnp.concatenate([ei[0], pad_e])
    edst = jnp.concatenate([ei[1], pad_e])
    scal = jnp.stack([ml, f, lf, ll], axis=1)
    scal = jnp.concatenate(
        [scal,
         jnp.concatenate([jnp.ones((NP - N, 1), jnp.float32),
                          jnp.zeros((NP - N, 3), jnp.float32)], axis=1)])
    z64 = jnp.zeros((NB, Q), jnp.float32)
    z16 = jnp.zeros((NB, 16), jnp.float32)
    ones16 = jnp.ones((C, 16), jnp.float32)

    we0, we1, we2, we3, dsrc, ddst = _embdeg(
        tq[0], tq[1], tq[2], tq[3], wpad, nids, esrc, edst, z64, z16, ones16)
    x = _tc1(we0, we1, we2, we3, scal, dsrc, W1)
    m = _msg(x[0], x[1], x[2], x[3], esrc, edst, z64)
    y = _tc2(m[0], m[1], m[2], m[3], dsrc, ddst, b1.reshape(1, D), W2)
    n = _msg(y[0], y[1], y[2], y[3], esrc, edst, z64)
    h, hg = _tc3(n[0], n[1], n[2], n[3], ddst, b2.reshape(1, D))
    return h[:N], hg
